# Initial kernel scaffold; baseline (speedup 1.0000x reference)
import jax
import jax.numpy as jnp
from jax.experimental import pallas as pl


def kernel(x, lmbda):
    raise NotImplementedError("write your pallas kernel here")


# speedup vs baseline: 4.0074x; 4.0074x over previous
import jax
import jax.numpy as jnp
from jax.experimental import pallas as pl

_N, _D = 65536, 512
_BLOCK_ROWS = 2048


def _yj_body(x_ref, lm_ref, o_ref):
    x = x_ref[...]
    lm = lm_ref[...]
    pos = x >= 0.0
    t = jnp.log(1.0 + jnp.abs(x))
    c = jnp.where(pos, lm, 2.0 - lm)
    czero = c == 0.0
    c_safe = jnp.where(czero, 1.0, c)
    r = jnp.where(czero, t, (jnp.exp(c * t) - 1.0) / c_safe)
    o_ref[...] = jnp.where(pos, r, -r)


def kernel(x, lmbda):
    n, d = x.shape
    lm2 = lmbda.reshape(1, d)
    grid = (n // _BLOCK_ROWS,)
    return pl.pallas_call(
        _yj_body,
        grid=grid,
        in_specs=[
            pl.BlockSpec((_BLOCK_ROWS, d), lambda i: (i, 0)),
            pl.BlockSpec((1, d), lambda i: (0, 0)),
        ],
        out_specs=pl.BlockSpec((_BLOCK_ROWS, d), lambda i: (i, 0)),
        out_shape=jax.ShapeDtypeStruct((n, d), x.dtype),
    )(x, lm2)


# speedup vs baseline: 4.2780x; 1.0675x over previous
import jax
import jax.numpy as jnp
from jax.experimental import pallas as pl

_N, _D = 65536, 512
_BLOCK_ROWS = 2048


_LN2 = 0.6931471805599453


def _yj_body(x_ref, lm_ref, o_ref):
    x = x_ref[...]
    lm = lm_ref[...]
    p1 = lm
    p2 = 2.0 - lm
    q1 = jnp.where(lm == 0.0, _LN2, 1.0 / jnp.where(lm == 0.0, 1.0, lm))
    q2 = jnp.where(lm == 2.0, -_LN2, -1.0 / jnp.where(lm == 2.0, 1.0, p2))
    pos = x >= 0.0
    t2 = jnp.log2(1.0 + jnp.abs(x))
    c = jnp.where(pos, p1, p2)
    em1 = jnp.exp2(c * t2) - 1.0
    a = jnp.where(c == 0.0, t2, em1)
    m = jnp.where(pos, q1, q2)
    o_ref[...] = a * m


def kernel(x, lmbda):
    n, d = x.shape
    lm2 = lmbda.reshape(1, d)
    grid = (n // _BLOCK_ROWS,)
    return pl.pallas_call(
        _yj_body,
        grid=grid,
        in_specs=[
            pl.BlockSpec((_BLOCK_ROWS, d), lambda i: (i, 0)),
            pl.BlockSpec((1, d), lambda i: (0, 0)),
        ],
        out_specs=pl.BlockSpec((_BLOCK_ROWS, d), lambda i: (i, 0)),
        out_shape=jax.ShapeDtypeStruct((n, d), x.dtype),
    )(x, lm2)


# speedup vs baseline: 4.5797x; 1.0705x over previous
import jax
import jax.numpy as jnp
from jax.experimental import pallas as pl

_N, _D = 65536, 512
_BLOCK_ROWS = 4096


_LN2 = 0.6931471805599453


def _yj_body(x_ref, lm_ref, o_ref):
    x = x_ref[...]
    lm = lm_ref[...]
    p1 = lm
    p2 = 2.0 - lm
    q1 = jnp.where(lm == 0.0, _LN2, 1.0 / jnp.where(lm == 0.0, 1.0, lm))
    q2 = jnp.where(lm == 2.0, -_LN2, -1.0 / jnp.where(lm == 2.0, 1.0, p2))
    pos = x >= 0.0
    t2 = jnp.log2(1.0 + jnp.abs(x))
    c = jnp.where(pos, p1, p2)
    em1 = jnp.exp2(c * t2) - 1.0
    a = jnp.where(c == 0.0, t2, em1)
    m = jnp.where(pos, q1, q2)
    o_ref[...] = a * m


def kernel(x, lmbda):
    n, d = x.shape
    lm2 = lmbda.reshape(1, d)
    grid = (n // _BLOCK_ROWS,)
    return pl.pallas_call(
        _yj_body,
        grid=grid,
        in_specs=[
            pl.BlockSpec((_BLOCK_ROWS, d), lambda i: (i, 0)),
            pl.BlockSpec((1, d), lambda i: (0, 0)),
        ],
        out_specs=pl.BlockSpec((_BLOCK_ROWS, d), lambda i: (i, 0)),
        out_shape=jax.ShapeDtypeStruct((n, d), x.dtype),
    )(x, lm2)
